# baseline (speedup 1.0000x reference)
import functools

import jax
import jax.numpy as jnp
from jax import lax
from jax.experimental import pallas as pl
from jax.experimental.pallas import tpu as pltpu
from jax.experimental.pallas import tpu_sc as plsc

N_VARS = 6000
N_CONS = 4000
E = 320000
D = 128
B = 64

NUM_SUBCORES = 16
CHUNK = 128
NCHUNKS = 160
EDGES_PER_TILE = NCHUNKS * CHUNK
E_PAD = NUM_SUBCORES * EDGES_PER_TILE

NVP = 6016
NCP = 4096
ROWS_V = NVP // NUM_SUBCORES
ROWS_C = NCP // NUM_SUBCORES

_MESH = plsc.VectorSubcoreMesh(core_axis_name="c", subcore_axis_name="s")


def _fill(ref, nrows, value):
    val = jnp.full((16,), value, jnp.float32)

    def st(r, carry):
        for k in range(D // 16):
            ref[r, pl.ds(k * 16, 16)] = val
        return carry

    lax.fori_loop(0, nrows, st, 0)


def _zero_stripe(sh_ref, s, nrows, zrows):
    base = s * nrows
    nfull, rem = divmod(nrows, CHUNK)
    for k in range(nfull):
        pltpu.sync_copy(zrows, sh_ref.at[pl.ds(base + k * CHUNK, CHUNK)])
    if rem:
        pltpu.sync_copy(zrows.at[pl.ds(0, rem)],
                        sh_ref.at[pl.ds(base + nfull * CHUNK, rem)])


def _emit_stripe(sh_ref, out_ref, s, nrows):
    sl = pl.ds(s * nrows, nrows)
    pltpu.sync_copy(sh_ref.at[sl], out_ref.at[sl])


@functools.partial(
    pl.kernel,
    out_type=(
        jax.ShapeDtypeStruct((NVP, D), jnp.float32),
        jax.ShapeDtypeStruct((NCP, D), jnp.float32),
    ),
    mesh=_MESH,
    scratch_types=[
        pltpu.VMEM((NCHUNKS, CHUNK), jnp.int32),
        pltpu.VMEM((NCHUNKS, CHUNK), jnp.int32),
        pltpu.VMEM((2, CHUNK, D), jnp.float32),
        pltpu.VMEM_SHARED((NVP, D), jnp.float32),
        pltpu.SemaphoreType.DMA((2,)),
        pltpu.SemaphoreType.DMA((2,)),
    ],
)
def _sc_agg(xv_hbm, xc_hbm, src_hbm, dst_hbm,
            aggv_out, aggc_out, gidx, sidx, rows, agg_sh, gsem, ssem):
    c = lax.axis_index("c")
    s = lax.axis_index("s")

    def setup(gidx_hbm, sidx_hbm, nrows):
        _fill(rows.at[0], CHUNK, 0.0)
        _zero_stripe(agg_sh, s, nrows, rows.at[0])
        pltpu.sync_copy(gidx_hbm.at[s], gidx)
        pltpu.sync_copy(sidx_hbm.at[s], sidx)

    @pl.when(c == 0)
    def _():
        setup(dst_hbm, src_hbm, ROWS_V)

    @pl.when(c == 1)
    def _():
        setup(src_hbm, dst_hbm, ROWS_C)

    plsc.subcore_barrier()

    def run(table_hbm):
        def g_start(b, j):
            pltpu.async_copy(table_hbm.at[gidx.at[j]], rows.at[b],
                             gsem.at[b])

        def g_wait(b, j):
            pltpu.make_async_copy(table_hbm.at[gidx.at[j]], rows.at[b],
                                  gsem.at[b]).wait()

        def s_start(b, j):
            pltpu.async_copy(rows.at[b], agg_sh.at[sidx.at[j]], ssem.at[b],
                             add=True)

        def s_wait(b, j):
            pltpu.make_async_copy(rows.at[b], agg_sh.at[sidx.at[j]],
                                  ssem.at[b]).wait()

        g_start(0, 0)

        def step(h, carry):
            j = h * 2
            g_start(1, j + 1)
            g_wait(0, j)
            s_start(0, j)
            g_wait(1, j + 1)
            s_wait(0, j)
            s_start(1, j + 1)

            @pl.when(h < NCHUNKS // 2 - 1)
            def _():
                g_start(0, j + 2)

            s_wait(1, j + 1)
            return carry

        lax.fori_loop(0, NCHUNKS // 2, step, 0)

    @pl.when(c == 0)
    def _():
        run(xc_hbm)

    @pl.when(c == 1)
    def _():
        run(xv_hbm)

    plsc.subcore_barrier()

    @pl.when(c == 0)
    def _():
        _emit_stripe(agg_sh, aggv_out, s, ROWS_V)

    @pl.when(c == 1)
    def _():
        _emit_stripe(agg_sh, aggc_out, s, ROWS_C)


@functools.partial(
    pl.kernel,
    out_type=(
        jax.ShapeDtypeStruct((NVP, D), jnp.float32),
        jax.ShapeDtypeStruct((NCP, D), jnp.float32),
    ),
    mesh=_MESH,
    scratch_types=[
        pltpu.VMEM((NCHUNKS, CHUNK), jnp.int32),
        pltpu.VMEM((CHUNK, D), jnp.float32),
        pltpu.VMEM((CHUNK, D), jnp.float32),
        pltpu.VMEM_SHARED((NVP, D), jnp.float32),
    ],
)
def _sc_counts(src_hbm, dst_hbm, cntv_out, cntc_out,
               sidx, ones_vm, zeros_vm, cnt_sh):
    c = lax.axis_index("c")
    s = lax.axis_index("s")

    def setup(sidx_hbm, nrows):
        _fill(zeros_vm, CHUNK, 0.0)
        _fill(ones_vm, CHUNK, 1.0)
        _zero_stripe(cnt_sh, s, nrows, zeros_vm)
        pltpu.sync_copy(sidx_hbm.at[s], sidx)

    @pl.when(c == 0)
    def _():
        setup(src_hbm, ROWS_V)

    @pl.when(c == 1)
    def _():
        setup(dst_hbm, ROWS_C)

    plsc.subcore_barrier()

    def step(j, carry):
        pltpu.sync_copy(ones_vm, cnt_sh.at[sidx.at[j]], add=True)
        return carry

    lax.fori_loop(0, NCHUNKS, step, 0)
    plsc.subcore_barrier()

    @pl.when(c == 0)
    def _():
        _emit_stripe(cnt_sh, cntv_out, s, ROWS_V)

    @pl.when(c == 1)
    def _():
        _emit_stripe(cnt_sh, cntc_out, s, ROWS_C)


def _layer_body(relu, aggv_ref, aggc_ref, cntv_ref, cntc_ref, xv_ref, xc_ref,
                wrv, wnv, bv, wrc, wnc, bc, hv_out, hc_out):
    def side(agg_ref, cnt_ref, n, npad, x_ref, wr, wn, b, out_ref):
        agg = agg_ref[0:n, :]
        cnt = cnt_ref[0:n, 0:1]
        mean = agg / jnp.maximum(cnt, 1.0)
        h = (jnp.dot(x_ref[0:n, :], wr[...],
                     preferred_element_type=jnp.float32)
             + jnp.dot(mean, wn[...], preferred_element_type=jnp.float32)
             + b[...])
        if relu:
            h = jnp.maximum(h, 0.0)
        out_ref[...] = jnp.concatenate(
            [h, jnp.zeros((npad - n, D), jnp.float32)], axis=0)

    side(aggv_ref, cntv_ref, N_VARS, NVP, xv_ref, wrv, wnv, bv, hv_out)
    side(aggc_ref, cntc_ref, N_CONS, NCP, xc_ref, wrc, wnc, bc, hc_out)


def _tc_layer(relu, aggv, aggc, cntv, cntc, xv, xc, wrv, wnv, bv, wrc, wnc,
              bc):
    return pl.pallas_call(
        functools.partial(_layer_body, relu),
        out_shape=(
            jax.ShapeDtypeStruct((NVP, D), jnp.float32),
            jax.ShapeDtypeStruct((NCP, D), jnp.float32),
        ),
    )(aggv, aggc, cntv, cntc, xv, xc, wrv, wnv, bv, wrc, wnc, bc)


def _pool_body(ov_ref, oc_ref, bv_ref, bc_ref,
               wp0, bp0, wp1, bp1, wh0, bh0, wh1, bh1, out_ref):
    def pool(batch_ref, x_ref, n):
        iota = lax.broadcasted_iota(jnp.int32, (B, n), 0)
        onehot = (batch_ref[...] == iota).astype(jnp.float32)
        ssum = jnp.dot(onehot, x_ref[0:n, :],
                       preferred_element_type=jnp.float32)
        cnt = jnp.sum(onehot, axis=1, keepdims=True)
        return ssum / jnp.maximum(cnt, 1.0)

    g_v = pool(bv_ref, ov_ref, N_VARS)
    g_c = pool(bc_ref, oc_ref, N_CONS)
    x = jnp.concatenate([g_v, g_c], axis=1)
    x = jnp.dot(x, wp0[...], preferred_element_type=jnp.float32) + bp0[...]
    x = jnp.maximum(x, 0.0)
    x = jnp.dot(x, wp1[...], preferred_element_type=jnp.float32) + bp1[...]
    h = jnp.dot(x, wh0[...], preferred_element_type=jnp.float32) + bh0[...]
    h = jnp.maximum(h, 0.0)
    out_ref[...] = (jnp.dot(h, wh1[...], preferred_element_type=jnp.float32)
                    + bh1[...])


def kernel(x_vars, x_cons, edge_index_vc, edge_index_cv, batch_vars,
           batch_cons, params):
    p = params
    pad = jnp.full((E_PAD - E,), N_VARS, jnp.int32)
    src = jnp.concatenate([edge_index_vc[0], pad]).reshape(
        NUM_SUBCORES, NCHUNKS, CHUNK)
    pad_c = jnp.full((E_PAD - E,), N_CONS, jnp.int32)
    dst = jnp.concatenate([edge_index_vc[1], pad_c]).reshape(
        NUM_SUBCORES, NCHUNKS, CHUNK)
    xv_pad = jnp.concatenate(
        [x_vars, jnp.zeros((NVP - N_VARS, D), jnp.float32)], axis=0)
    xc_pad = jnp.concatenate(
        [x_cons, jnp.zeros((NCP - N_CONS, D), jnp.float32)], axis=0)

    def b2(name):
        return p[name].reshape(1, -1)

    cntv, cntc = _sc_counts(src, dst)
    aggv, aggc = _sc_agg(xv_pad, xc_pad, src, dst)
    h_v, h_c = _tc_layer(True, aggv, aggc, cntv, cntc, xv_pad, xc_pad,
                         p['Wr_v0'], p['Wn_v0'], b2('b_v0'),
                         p['Wr_c0'], p['Wn_c0'], b2('b_c0'))
    aggv1, aggc1 = _sc_agg(h_v, h_c, src, dst)
    o_v, o_c = _tc_layer(False, aggv1, aggc1, cntv, cntc, h_v, h_c,
                         p['Wr_v1'], p['Wn_v1'], b2('b_v1'),
                         p['Wr_c1'], p['Wn_c1'], b2('b_c1'))

    wh0 = jnp.concatenate([p['Wh0_0'], p['Wh1_0'], p['Wh2_0']], axis=1)
    bh0 = jnp.concatenate([p['bh0_0'], p['bh1_0'], p['bh2_0']]).reshape(1, -1)
    wh1 = jax.scipy.linalg.block_diag(p['Wh0_1'], p['Wh1_1'], p['Wh2_1'])
    bh1 = jnp.concatenate([p['bh0_1'], p['bh1_1'], p['bh2_1']]).reshape(1, -1)

    out = pl.pallas_call(
        _pool_body,
        out_shape=jax.ShapeDtypeStruct((B, 3), jnp.float32),
    )(o_v, o_c, batch_vars.reshape(1, N_VARS), batch_cons.reshape(1, N_CONS),
      p['Wp0'], b2('bp0'), p['Wp1'], b2('bp1'), wh0, bh0, wh1, bh1)
    return out


# speedup vs baseline: 1.0168x; 1.0168x over previous
import functools

import jax
import jax.numpy as jnp
from jax import lax
from jax.experimental import pallas as pl
from jax.experimental.pallas import tpu as pltpu
from jax.experimental.pallas import tpu_sc as plsc

N_VARS = 6000
N_CONS = 4000
E = 320000
D = 128
B = 64

NUM_SUBCORES = 16
CHUNK = 128
NCHUNKS = 160
EDGES_PER_TILE = NCHUNKS * CHUNK
E_PAD = NUM_SUBCORES * EDGES_PER_TILE

NVP = 6016
NCP = 4096
ROWS_V = NVP // NUM_SUBCORES
ROWS_C = NCP // NUM_SUBCORES

_MESH = plsc.VectorSubcoreMesh(core_axis_name="c", subcore_axis_name="s")


def _fill(ref, nrows, value):
    val = jnp.full((16,), value, jnp.float32)

    def st(r, carry):
        for k in range(D // 16):
            ref[r, pl.ds(k * 16, 16)] = val
        return carry

    lax.fori_loop(0, nrows, st, 0)


def _zero_stripe(sh_ref, s, nrows, zrows):
    base = s * nrows
    nfull, rem = divmod(nrows, CHUNK)
    for k in range(nfull):
        pltpu.sync_copy(zrows, sh_ref.at[pl.ds(base + k * CHUNK, CHUNK)])
    if rem:
        pltpu.sync_copy(zrows.at[pl.ds(0, rem)],
                        sh_ref.at[pl.ds(base + nfull * CHUNK, rem)])


def _emit_stripe(sh_ref, out_ref, s, nrows):
    sl = pl.ds(s * nrows, nrows)
    pltpu.sync_copy(sh_ref.at[sl], out_ref.at[sl])


@functools.partial(
    pl.kernel,
    out_type=(
        jax.ShapeDtypeStruct((NVP, D), jnp.float32),
        jax.ShapeDtypeStruct((NCP, D), jnp.float32),
    ),
    mesh=_MESH,
    scratch_types=[
        pltpu.VMEM((NCHUNKS, CHUNK), jnp.int32),
        pltpu.VMEM((NCHUNKS, CHUNK), jnp.int32),
        pltpu.VMEM((2, CHUNK, D), jnp.float32),
        pltpu.VMEM_SHARED((NVP, D), jnp.float32),
        pltpu.SemaphoreType.DMA((2,)),
        pltpu.SemaphoreType.DMA((2,)),
    ],
)
def _sc_agg(xv_hbm, xc_hbm, src_hbm, dst_hbm,
            aggv_out, aggc_out, gidx, sidx, rows, agg_sh, gsem, ssem):
    c = lax.axis_index("c")
    s = lax.axis_index("s")

    def setup(gidx_hbm, sidx_hbm, nrows):
        _fill(rows.at[0], CHUNK, 0.0)
        _zero_stripe(agg_sh, s, nrows, rows.at[0])
        pltpu.sync_copy(gidx_hbm.at[s], gidx)
        pltpu.sync_copy(sidx_hbm.at[s], sidx)

    @pl.when(c == 0)
    def _():
        setup(dst_hbm, src_hbm, ROWS_V)

    @pl.when(c == 1)
    def _():
        setup(src_hbm, dst_hbm, ROWS_C)

    plsc.subcore_barrier()

    def run(table_hbm):
        H = CHUNK // 2

        def g_start(b, j):
            pltpu.async_copy(table_hbm.at[gidx.at[j, pl.ds(0, H)]],
                             rows.at[b, pl.ds(0, H)], gsem.at[b])
            pltpu.async_copy(table_hbm.at[gidx.at[j, pl.ds(H, H)]],
                             rows.at[b, pl.ds(H, H)], ssem.at[b])

        def g_wait(b, j):
            pltpu.make_async_copy(table_hbm.at[gidx.at[j, pl.ds(0, H)]],
                                  rows.at[b, pl.ds(0, H)], gsem.at[b]).wait()
            pltpu.make_async_copy(table_hbm.at[gidx.at[j, pl.ds(H, H)]],
                                  rows.at[b, pl.ds(H, H)], ssem.at[b]).wait()

        def scat(b, j):
            pltpu.sync_copy(rows.at[b], agg_sh.at[sidx.at[j]], add=True)

        g_start(0, 0)

        def step(h, carry):
            j = h * 2
            g_start(1, j + 1)
            g_wait(0, j)
            scat(0, j)

            @pl.when(h < NCHUNKS // 2 - 1)
            def _():
                g_start(0, j + 2)

            g_wait(1, j + 1)
            scat(1, j + 1)
            return carry

        lax.fori_loop(0, NCHUNKS // 2, step, 0)

    @pl.when(c == 0)
    def _():
        run(xc_hbm)

    @pl.when(c == 1)
    def _():
        run(xv_hbm)

    plsc.subcore_barrier()

    @pl.when(c == 0)
    def _():
        _emit_stripe(agg_sh, aggv_out, s, ROWS_V)

    @pl.when(c == 1)
    def _():
        _emit_stripe(agg_sh, aggc_out, s, ROWS_C)


@functools.partial(
    pl.kernel,
    out_type=(
        jax.ShapeDtypeStruct((NVP, D), jnp.float32),
        jax.ShapeDtypeStruct((NCP, D), jnp.float32),
    ),
    mesh=_MESH,
    scratch_types=[
        pltpu.VMEM((NCHUNKS, CHUNK), jnp.int32),
        pltpu.VMEM((CHUNK, D), jnp.float32),
        pltpu.VMEM((CHUNK, D), jnp.float32),
        pltpu.VMEM_SHARED((NVP, D), jnp.float32),
    ],
)
def _sc_counts(src_hbm, dst_hbm, cntv_out, cntc_out,
               sidx, ones_vm, zeros_vm, cnt_sh):
    c = lax.axis_index("c")
    s = lax.axis_index("s")

    def setup(sidx_hbm, nrows):
        _fill(zeros_vm, CHUNK, 0.0)
        _fill(ones_vm, CHUNK, 1.0)
        _zero_stripe(cnt_sh, s, nrows, zeros_vm)
        pltpu.sync_copy(sidx_hbm.at[s], sidx)

    @pl.when(c == 0)
    def _():
        setup(src_hbm, ROWS_V)

    @pl.when(c == 1)
    def _():
        setup(dst_hbm, ROWS_C)

    plsc.subcore_barrier()

    def step(j, carry):
        pltpu.sync_copy(ones_vm, cnt_sh.at[sidx.at[j]], add=True)
        return carry

    lax.fori_loop(0, NCHUNKS, step, 0)
    plsc.subcore_barrier()

    @pl.when(c == 0)
    def _():
        _emit_stripe(cnt_sh, cntv_out, s, ROWS_V)

    @pl.when(c == 1)
    def _():
        _emit_stripe(cnt_sh, cntc_out, s, ROWS_C)


def _layer_body(relu, aggv_ref, aggc_ref, cntv_ref, cntc_ref, xv_ref, xc_ref,
                wrv, wnv, bv, wrc, wnc, bc, hv_out, hc_out):
    def side(agg_ref, cnt_ref, n, npad, x_ref, wr, wn, b, out_ref):
        agg = agg_ref[0:n, :]
        cnt = cnt_ref[0:n, 0:1]
        mean = agg / jnp.maximum(cnt, 1.0)
        h = (jnp.dot(x_ref[0:n, :], wr[...],
                     preferred_element_type=jnp.float32)
             + jnp.dot(mean, wn[...], preferred_element_type=jnp.float32)
             + b[...])
        if relu:
            h = jnp.maximum(h, 0.0)
        out_ref[...] = jnp.concatenate(
            [h, jnp.zeros((npad - n, D), jnp.float32)], axis=0)

    side(aggv_ref, cntv_ref, N_VARS, NVP, xv_ref, wrv, wnv, bv, hv_out)
    side(aggc_ref, cntc_ref, N_CONS, NCP, xc_ref, wrc, wnc, bc, hc_out)


def _tc_layer(relu, aggv, aggc, cntv, cntc, xv, xc, wrv, wnv, bv, wrc, wnc,
              bc):
    return pl.pallas_call(
        functools.partial(_layer_body, relu),
        out_shape=(
            jax.ShapeDtypeStruct((NVP, D), jnp.float32),
            jax.ShapeDtypeStruct((NCP, D), jnp.float32),
        ),
    )(aggv, aggc, cntv, cntc, xv, xc, wrv, wnv, bv, wrc, wnc, bc)


def _pool_body(ov_ref, oc_ref, bv_ref, bc_ref,
               wp0, bp0, wp1, bp1, wh0, bh0, wh1, bh1, out_ref):
    def pool(batch_ref, x_ref, n):
        iota = lax.broadcasted_iota(jnp.int32, (B, n), 0)
        onehot = (batch_ref[...] == iota).astype(jnp.float32)
        ssum = jnp.dot(onehot, x_ref[0:n, :],
                       preferred_element_type=jnp.float32)
        cnt = jnp.sum(onehot, axis=1, keepdims=True)
        return ssum / jnp.maximum(cnt, 1.0)

    g_v = pool(bv_ref, ov_ref, N_VARS)
    g_c = pool(bc_ref, oc_ref, N_CONS)
    x = jnp.concatenate([g_v, g_c], axis=1)
    x = jnp.dot(x, wp0[...], preferred_element_type=jnp.float32) + bp0[...]
    x = jnp.maximum(x, 0.0)
    x = jnp.dot(x, wp1[...], preferred_element_type=jnp.float32) + bp1[...]
    h = jnp.dot(x, wh0[...], preferred_element_type=jnp.float32) + bh0[...]
    h = jnp.maximum(h, 0.0)
    out_ref[...] = (jnp.dot(h, wh1[...], preferred_element_type=jnp.float32)
                    + bh1[...])


def kernel(x_vars, x_cons, edge_index_vc, edge_index_cv, batch_vars,
           batch_cons, params):
    p = params
    pad = jnp.full((E_PAD - E,), N_VARS, jnp.int32)
    src = jnp.concatenate([edge_index_vc[0], pad]).reshape(
        NUM_SUBCORES, NCHUNKS, CHUNK)
    pad_c = jnp.full((E_PAD - E,), N_CONS, jnp.int32)
    dst = jnp.concatenate([edge_index_vc[1], pad_c]).reshape(
        NUM_SUBCORES, NCHUNKS, CHUNK)
    xv_pad = jnp.concatenate(
        [x_vars, jnp.zeros((NVP - N_VARS, D), jnp.float32)], axis=0)
    xc_pad = jnp.concatenate(
        [x_cons, jnp.zeros((NCP - N_CONS, D), jnp.float32)], axis=0)

    def b2(name):
        return p[name].reshape(1, -1)

    cntv, cntc = _sc_counts(src, dst)
    aggv, aggc = _sc_agg(xv_pad, xc_pad, src, dst)
    h_v, h_c = _tc_layer(True, aggv, aggc, cntv, cntc, xv_pad, xc_pad,
                         p['Wr_v0'], p['Wn_v0'], b2('b_v0'),
                         p['Wr_c0'], p['Wn_c0'], b2('b_c0'))
    aggv1, aggc1 = _sc_agg(h_v, h_c, src, dst)
    o_v, o_c = _tc_layer(False, aggv1, aggc1, cntv, cntc, h_v, h_c,
                         p['Wr_v1'], p['Wn_v1'], b2('b_v1'),
                         p['Wr_c1'], p['Wn_c1'], b2('b_c1'))

    wh0 = jnp.concatenate([p['Wh0_0'], p['Wh1_0'], p['Wh2_0']], axis=1)
    bh0 = jnp.concatenate([p['bh0_0'], p['bh1_0'], p['bh2_0']]).reshape(1, -1)
    wh1 = jax.scipy.linalg.block_diag(p['Wh0_1'], p['Wh1_1'], p['Wh2_1'])
    bh1 = jnp.concatenate([p['bh0_1'], p['bh1_1'], p['bh2_1']]).reshape(1, -1)

    out = pl.pallas_call(
        _pool_body,
        out_shape=jax.ShapeDtypeStruct((B, 3), jnp.float32),
    )(o_v, o_c, batch_vars.reshape(1, N_VARS), batch_cons.reshape(1, N_CONS),
      p['Wp0'], b2('bp0'), p['Wp1'], b2('bp1'), wh0, bh0, wh1, bh1)
    return out


# speedup vs baseline: 1.0203x; 1.0035x over previous
import functools

import jax
import jax.numpy as jnp
from jax import lax
from jax.experimental import pallas as pl
from jax.experimental.pallas import tpu as pltpu
from jax.experimental.pallas import tpu_sc as plsc

N_VARS = 6000
N_CONS = 4000
E = 320000
D = 128
B = 64

NUM_SUBCORES = 16
CHUNK = 128
NCHUNKS = 160
EDGES_PER_TILE = NCHUNKS * CHUNK
E_PAD = NUM_SUBCORES * EDGES_PER_TILE

NVP = 6016
NCP = 4096
ROWS_V = NVP // NUM_SUBCORES
ROWS_C = NCP // NUM_SUBCORES

_MESH = plsc.VectorSubcoreMesh(core_axis_name="c", subcore_axis_name="s")


def _fill(ref, nrows, value):
    val = jnp.full((16,), value, jnp.float32)

    def st(r, carry):
        for k in range(D // 16):
            ref[r, pl.ds(k * 16, 16)] = val
        return carry

    lax.fori_loop(0, nrows, st, 0)


def _zero_stripe(sh_ref, s, nrows, zrows):
    base = s * nrows
    nfull, rem = divmod(nrows, CHUNK)
    for k in range(nfull):
        pltpu.sync_copy(zrows, sh_ref.at[pl.ds(base + k * CHUNK, CHUNK)])
    if rem:
        pltpu.sync_copy(zrows.at[pl.ds(0, rem)],
                        sh_ref.at[pl.ds(base + nfull * CHUNK, rem)])


def _emit_stripe(sh_ref, out_ref, s, nrows):
    sl = pl.ds(s * nrows, nrows)
    pltpu.sync_copy(sh_ref.at[sl], out_ref.at[sl])


@functools.partial(
    pl.kernel,
    out_type=(
        jax.ShapeDtypeStruct((NVP, D), jnp.float32),
        jax.ShapeDtypeStruct((NCP, D), jnp.float32),
    ),
    mesh=_MESH,
    scratch_types=[
        pltpu.VMEM((NCHUNKS, CHUNK), jnp.int32),
        pltpu.VMEM((NCHUNKS, CHUNK), jnp.int32),
        pltpu.VMEM((2, CHUNK, D), jnp.float32),
        pltpu.VMEM_SHARED((NVP, D), jnp.float32),
        pltpu.SemaphoreType.DMA((2,)),
        pltpu.SemaphoreType.DMA((2,)),
    ],
)
def _sc_agg(xv_hbm, xc_hbm, src_hbm, dst_hbm,
            aggv_out, aggc_out, gidx, sidx, rows, agg_sh, gsem, ssem):
    c = lax.axis_index("c")
    s = lax.axis_index("s")

    def setup(gidx_hbm, sidx_hbm, nrows):
        _fill(rows.at[0], CHUNK, 0.0)
        _zero_stripe(agg_sh, s, nrows, rows.at[0])
        pltpu.sync_copy(gidx_hbm.at[s], gidx)
        pltpu.sync_copy(sidx_hbm.at[s], sidx)

    @pl.when(c == 0)
    def _():
        setup(dst_hbm, src_hbm, ROWS_V)

    @pl.when(c == 1)
    def _():
        setup(src_hbm, dst_hbm, ROWS_C)

    plsc.subcore_barrier()

    def run(table_hbm):
        def g_start(b, j):
            pltpu.async_copy(table_hbm.at[gidx.at[j]], rows.at[b],
                             gsem.at[b])

        def g_wait(b, j):
            pltpu.make_async_copy(table_hbm.at[gidx.at[j]], rows.at[b],
                                  gsem.at[b]).wait()

        def scat(b, j):
            pltpu.sync_copy(rows.at[b], agg_sh.at[sidx.at[j]], add=True)

        g_start(0, 0)

        def step(h, carry):
            j = h * 2
            g_start(1, j + 1)
            g_wait(0, j)
            scat(0, j)

            @pl.when(h < NCHUNKS // 2 - 1)
            def _():
                g_start(0, j + 2)

            g_wait(1, j + 1)
            scat(1, j + 1)
            return carry

        lax.fori_loop(0, NCHUNKS // 2, step, 0)

    @pl.when(c == 0)
    def _():
        run(xc_hbm)

    @pl.when(c == 1)
    def _():
        run(xv_hbm)

    plsc.subcore_barrier()

    @pl.when(c == 0)
    def _():
        _emit_stripe(agg_sh, aggv_out, s, ROWS_V)

    @pl.when(c == 1)
    def _():
        _emit_stripe(agg_sh, aggc_out, s, ROWS_C)


@functools.partial(
    pl.kernel,
    out_type=(
        jax.ShapeDtypeStruct((NVP, D), jnp.float32),
        jax.ShapeDtypeStruct((NCP, D), jnp.float32),
    ),
    mesh=_MESH,
    scratch_types=[
        pltpu.VMEM((NCHUNKS, CHUNK), jnp.int32),
        pltpu.VMEM((CHUNK, D), jnp.float32),
        pltpu.VMEM((CHUNK, D), jnp.float32),
        pltpu.VMEM_SHARED((NVP, D), jnp.float32),
    ],
)
def _sc_counts(src_hbm, dst_hbm, cntv_out, cntc_out,
               sidx, ones_vm, zeros_vm, cnt_sh):
    c = lax.axis_index("c")
    s = lax.axis_index("s")

    def setup(sidx_hbm, nrows):
        _fill(zeros_vm, CHUNK, 0.0)
        _fill(ones_vm, CHUNK, 1.0)
        _zero_stripe(cnt_sh, s, nrows, zeros_vm)
        pltpu.sync_copy(sidx_hbm.at[s], sidx)

    @pl.when(c == 0)
    def _():
        setup(src_hbm, ROWS_V)

    @pl.when(c == 1)
    def _():
        setup(dst_hbm, ROWS_C)

    plsc.subcore_barrier()

    def step(j, carry):
        pltpu.sync_copy(ones_vm, cnt_sh.at[sidx.at[j]], add=True)
        return carry

    lax.fori_loop(0, NCHUNKS, step, 0)
    plsc.subcore_barrier()

    @pl.when(c == 0)
    def _():
        _emit_stripe(cnt_sh, cntv_out, s, ROWS_V)

    @pl.when(c == 1)
    def _():
        _emit_stripe(cnt_sh, cntc_out, s, ROWS_C)


def _layer_body(relu, aggv_ref, aggc_ref, cntv_ref, cntc_ref, xv_ref, xc_ref,
                wrv, wnv, bv, wrc, wnc, bc, hv_out, hc_out):
    def side(agg_ref, cnt_ref, n, npad, x_ref, wr, wn, b, out_ref):
        agg = agg_ref[0:n, :]
        cnt = cnt_ref[0:n, 0:1]
        mean = agg / jnp.maximum(cnt, 1.0)
        h = (jnp.dot(x_ref[0:n, :], wr[...],
                     preferred_element_type=jnp.float32)
             + jnp.dot(mean, wn[...], preferred_element_type=jnp.float32)
             + b[...])
        if relu:
            h = jnp.maximum(h, 0.0)
        out_ref[...] = jnp.concatenate(
            [h, jnp.zeros((npad - n, D), jnp.float32)], axis=0)

    side(aggv_ref, cntv_ref, N_VARS, NVP, xv_ref, wrv, wnv, bv, hv_out)
    side(aggc_ref, cntc_ref, N_CONS, NCP, xc_ref, wrc, wnc, bc, hc_out)


def _tc_layer(relu, aggv, aggc, cntv, cntc, xv, xc, wrv, wnv, bv, wrc, wnc,
              bc):
    return pl.pallas_call(
        functools.partial(_layer_body, relu),
        out_shape=(
            jax.ShapeDtypeStruct((NVP, D), jnp.float32),
            jax.ShapeDtypeStruct((NCP, D), jnp.float32),
        ),
    )(aggv, aggc, cntv, cntc, xv, xc, wrv, wnv, bv, wrc, wnc, bc)


def _pool_body(ov_ref, oc_ref, bv_ref, bc_ref,
               wp0, bp0, wp1, bp1, wh0, bh0, wh1, bh1, out_ref):
    def pool(batch_ref, x_ref, n):
        iota = lax.broadcasted_iota(jnp.int32, (B, n), 0)
        onehot = (batch_ref[...] == iota).astype(jnp.float32)
        ssum = jnp.dot(onehot, x_ref[0:n, :],
                       preferred_element_type=jnp.float32)
        cnt = jnp.sum(onehot, axis=1, keepdims=True)
        return ssum / jnp.maximum(cnt, 1.0)

    g_v = pool(bv_ref, ov_ref, N_VARS)
    g_c = pool(bc_ref, oc_ref, N_CONS)
    x = jnp.concatenate([g_v, g_c], axis=1)
    x = jnp.dot(x, wp0[...], preferred_element_type=jnp.float32) + bp0[...]
    x = jnp.maximum(x, 0.0)
    x = jnp.dot(x, wp1[...], preferred_element_type=jnp.float32) + bp1[...]
    h = jnp.dot(x, wh0[...], preferred_element_type=jnp.float32) + bh0[...]
    h = jnp.maximum(h, 0.0)
    out_ref[...] = (jnp.dot(h, wh1[...], preferred_element_type=jnp.float32)
                    + bh1[...])


def kernel(x_vars, x_cons, edge_index_vc, edge_index_cv, batch_vars,
           batch_cons, params):
    p = params
    pad = jnp.full((E_PAD - E,), N_VARS, jnp.int32)
    src = jnp.concatenate([edge_index_vc[0], pad]).reshape(
        NUM_SUBCORES, NCHUNKS, CHUNK)
    pad_c = jnp.full((E_PAD - E,), N_CONS, jnp.int32)
    dst = jnp.concatenate([edge_index_vc[1], pad_c]).reshape(
        NUM_SUBCORES, NCHUNKS, CHUNK)
    xv_pad = jnp.concatenate(
        [x_vars, jnp.zeros((NVP - N_VARS, D), jnp.float32)], axis=0)
    xc_pad = jnp.concatenate(
        [x_cons, jnp.zeros((NCP - N_CONS, D), jnp.float32)], axis=0)

    def b2(name):
        return p[name].reshape(1, -1)

    cntv, cntc = _sc_counts(src, dst)
    aggv, aggc = _sc_agg(xv_pad, xc_pad, src, dst)
    h_v, h_c = _tc_layer(True, aggv, aggc, cntv, cntc, xv_pad, xc_pad,
                         p['Wr_v0'], p['Wn_v0'], b2('b_v0'),
                         p['Wr_c0'], p['Wn_c0'], b2('b_c0'))
    aggv1, aggc1 = _sc_agg(h_v, h_c, src, dst)
    o_v, o_c = _tc_layer(False, aggv1, aggc1, cntv, cntc, h_v, h_c,
                         p['Wr_v1'], p['Wn_v1'], b2('b_v1'),
                         p['Wr_c1'], p['Wn_c1'], b2('b_c1'))

    wh0 = jnp.concatenate([p['Wh0_0'], p['Wh1_0'], p['Wh2_0']], axis=1)
    bh0 = jnp.concatenate([p['bh0_0'], p['bh1_0'], p['bh2_0']]).reshape(1, -1)
    wh1 = jax.scipy.linalg.block_diag(p['Wh0_1'], p['Wh1_1'], p['Wh2_1'])
    bh1 = jnp.concatenate([p['bh0_1'], p['bh1_1'], p['bh2_1']]).reshape(1, -1)

    out = pl.pallas_call(
        _pool_body,
        out_shape=jax.ShapeDtypeStruct((B, 3), jnp.float32),
    )(o_v, o_c, batch_vars.reshape(1, N_VARS), batch_cons.reshape(1, N_CONS),
      p['Wp0'], b2('bp0'), p['Wp1'], b2('bp1'), wh0, bh0, wh1, bh1)
    return out
